# Initial kernel scaffold; baseline (speedup 1.0000x reference)
import jax
import jax.numpy as jnp
from jax.experimental import pallas as pl


def kernel(x, edge_index, W1, b1, W2, b2, Wl, bl):
    raise NotImplementedError("write your pallas kernel here")


# speedup vs baseline: 17.8626x; 17.8626x over previous
import functools

import jax
import jax.numpy as jnp
from jax import lax
from jax.experimental import pallas as pl
from jax.experimental.pallas import tpu as pltpu
from jax.experimental.pallas import tpu_sc as plsc

N_SC_CORES = 2
N_SUBCORES = 16
N_WORKERS = N_SC_CORES * N_SUBCORES
EDGE_BLK = 128
LANES = 16


def _fill_rows(ref, rows, cols, value):
    v = jnp.full((LANES,), value, jnp.float32)
    per_row = cols // LANES

    def body(i, carry):
        r = i // per_row
        c = (i % per_row) * LANES
        ref[r, pl.ds(c, LANES)] = v
        return carry

    lax.fori_loop(0, rows * per_row, body, 0)


def _fill_flat(ref, n, value):
    v = jnp.full((LANES,), value, jnp.float32)

    def body(i, carry):
        ref[pl.ds(i * LANES, LANES)] = v
        return carry

    lax.fori_loop(0, n // LANES, body, 0)


def _make_degree(N, E):
    n_chunks = E // EDGE_BLK
    full_ch = 640
    tail = N - (N_SUBCORES - 1) * full_ch
    mesh = plsc.VectorSubcoreMesh(core_axis_name="c", subcore_axis_name="s")

    @functools.partial(
        pl.kernel,
        out_type=jax.ShapeDtypeStruct((N_SC_CORES * N,), jnp.float32),
        mesh=mesh,
        scratch_types=[
            pltpu.VMEM((EDGE_BLK,), jnp.int32),
            pltpu.VMEM((EDGE_BLK,), jnp.float32),
            pltpu.VMEM((full_ch,), jnp.float32),
            pltpu.VMEM_SHARED((N,), jnp.float32),
            pltpu.SemaphoreType.DMA,
        ],
    )
    def deg_kernel(dst_hbm, out_hbm, dst_v, ones_v, zeros_v, acc_sh, sem):
        del sem
        cid = lax.axis_index("c")
        sid = lax.axis_index("s")
        wid = cid * N_SUBCORES + sid
        _fill_flat(ones_v, EDGE_BLK, 1.0)
        _fill_flat(zeros_v, full_ch, 0.0)
        base = sid * full_ch

        @pl.when(sid < N_SUBCORES - 1)
        def _():
            pltpu.sync_copy(zeros_v, acc_sh.at[pl.ds(base, full_ch)])

        @pl.when(sid == N_SUBCORES - 1)
        def _():
            pltpu.sync_copy(zeros_v.at[pl.ds(0, tail)], acc_sh.at[pl.ds(base, tail)])

        plsc.subcore_barrier()
        nloc = (n_chunks - wid + N_WORKERS - 1) // N_WORKERS

        def body(j, carry):
            off = (wid + j * N_WORKERS) * EDGE_BLK
            pltpu.sync_copy(dst_hbm.at[pl.ds(off, EDGE_BLK)], dst_v)
            pltpu.sync_copy(ones_v, acc_sh.at[dst_v], add=True)
            return carry

        lax.fori_loop(0, nloc, body, 0)
        plsc.subcore_barrier()

        @pl.when(sid < N_SUBCORES - 1)
        def _():
            pltpu.sync_copy(acc_sh.at[pl.ds(base, full_ch)], zeros_v)
            pltpu.sync_copy(zeros_v, out_hbm.at[pl.ds(cid * N + base, full_ch)])

        @pl.when(sid == N_SUBCORES - 1)
        def _():
            pltpu.sync_copy(acc_sh.at[pl.ds(base, tail)], zeros_v.at[pl.ds(0, tail)])
            pltpu.sync_copy(zeros_v.at[pl.ds(0, tail)],
                            out_hbm.at[pl.ds(cid * N + base, tail)])

    return deg_kernel


def _make_agg(N, D, E):
    n_chunks = E // EDGE_BLK
    full_rows = 640
    tail_rows = N - (N_SUBCORES - 1) * full_rows
    bch = 80
    mesh = plsc.VectorSubcoreMesh(core_axis_name="c", subcore_axis_name="s")

    @functools.partial(
        pl.kernel,
        out_type=jax.ShapeDtypeStruct((N_SC_CORES, N, D), jnp.float32),
        mesh=mesh,
        scratch_types=[
            pltpu.VMEM((EDGE_BLK,), jnp.int32),
            pltpu.VMEM((EDGE_BLK,), jnp.int32),
            pltpu.VMEM((EDGE_BLK, D), jnp.float32),
            pltpu.VMEM((bch, D), jnp.float32),
            pltpu.VMEM_SHARED((N, D), jnp.float32),
            pltpu.SemaphoreType.DMA,
        ],
        compiler_params=pltpu.CompilerParams(use_tc_tiling_on_sc=False),
    )
    def agg_kernel(hs_hbm, src_hbm, dst_hbm, out_hbm,
                   src_v, dst_v, rows_v, zrows_v, acc_sh, sem):
        cid = lax.axis_index("c")
        sid = lax.axis_index("s")
        wid = cid * N_SUBCORES + sid
        _fill_rows(zrows_v, bch, D, 0.0)
        base = sid * full_rows
        nch = jnp.where(sid < N_SUBCORES - 1, full_rows // bch, tail_rows // bch)
        for k in range(full_rows // bch):
            @pl.when(k < nch)
            def _():
                pltpu.sync_copy(zrows_v, acc_sh.at[pl.ds(base + k * bch, bch)])
        plsc.subcore_barrier()

        nloc = (n_chunks - wid + N_WORKERS - 1) // N_WORKERS

        def body(j, carry):
            off = (wid + j * N_WORKERS) * EDGE_BLK
            pltpu.sync_copy(src_hbm.at[pl.ds(off, EDGE_BLK)], src_v)
            pltpu.sync_copy(dst_hbm.at[pl.ds(off, EDGE_BLK)], dst_v)
            pltpu.async_copy(hs_hbm.at[src_v], rows_v, sem).wait()
            pltpu.sync_copy(rows_v, acc_sh.at[dst_v], add=True)
            return carry

        lax.fori_loop(0, nloc, body, 0)
        plsc.subcore_barrier()
        for k in range(full_rows // bch):
            @pl.when(k < nch)
            def _():
                pltpu.sync_copy(acc_sh.at[pl.ds(base + k * bch, bch)], zrows_v)
                pltpu.sync_copy(zrows_v, out_hbm.at[cid, pl.ds(base + k * bch, bch)])

    return agg_kernel


def _mm1_call(x, W1, deg):
    N, C = x.shape
    H = W1.shape[1]
    BR = 1000

    def body(x_ref, w_ref, deg_ref, hs_ref, dinv_ref):
        d = lax.rsqrt(deg_ref[0] + deg_ref[1] + 1.0)
        h = jnp.dot(x_ref[...], w_ref[...], preferred_element_type=jnp.float32)
        hs_ref[...] = h * d
        dinv_ref[...] = d

    return pl.pallas_call(
        body,
        grid=(N // BR,),
        in_specs=[
            pl.BlockSpec((BR, C), lambda i: (i, 0)),
            pl.BlockSpec((C, H), lambda i: (0, 0)),
            pl.BlockSpec((2, BR, 1), lambda i: (0, i, 0)),
        ],
        out_specs=[
            pl.BlockSpec((BR, H), lambda i: (i, 0)),
            pl.BlockSpec((BR, 1), lambda i: (i, 0)),
        ],
        out_shape=[
            jax.ShapeDtypeStruct((N, H), jnp.float32),
            jax.ShapeDtypeStruct((N, 1), jnp.float32),
        ],
    )(x, W1, deg)


def _mid_call(agg1, hs1, dinv, b1, W2):
    N, H = hs1.shape
    H2 = W2.shape[1]
    BR = 1000

    def body(a0_ref, a1_ref, hs_ref, d_ref, b_ref, w_ref, o_ref):
        a = a0_ref[0] + a1_ref[0]
        d = d_ref[...]
        z = d * (a + hs_ref[...]) + b_ref[...]
        r = jnp.maximum(z, 0.0)
        o_ref[...] = d * jnp.dot(r, w_ref[...], preferred_element_type=jnp.float32)

    return pl.pallas_call(
        body,
        grid=(N // BR,),
        in_specs=[
            pl.BlockSpec((1, BR, H), lambda i: (0, i, 0)),
            pl.BlockSpec((1, BR, H), lambda i: (1, i, 0)),
            pl.BlockSpec((BR, H), lambda i: (i, 0)),
            pl.BlockSpec((BR, 1), lambda i: (i, 0)),
            pl.BlockSpec((1, H), lambda i: (0, 0)),
            pl.BlockSpec((H, H2), lambda i: (0, 0)),
        ],
        out_specs=pl.BlockSpec((BR, H2), lambda i: (i, 0)),
        out_shape=jax.ShapeDtypeStruct((N, H2), jnp.float32),
    )(agg1, agg1, hs1, dinv, b1, W2)


def _final_call(agg2, hs2, dinv, b2, Wl, bl):
    N, H2 = hs2.shape
    O = Wl.shape[1]
    BR = 1000

    def body(a0_ref, a1_ref, hs_ref, d_ref, b_ref, w_ref, bl_ref, o_ref):
        a = a0_ref[0] + a1_ref[0]
        d = d_ref[...]
        z = d * (a + hs_ref[...]) + b_ref[...]
        logits = jnp.dot(z, w_ref[...], preferred_element_type=jnp.float32)
        logits = logits + bl_ref[...]
        m = jnp.max(logits, axis=1, keepdims=True)
        lse = jnp.log(jnp.sum(jnp.exp(logits - m), axis=1, keepdims=True)) + m
        o_ref[...] = logits - lse

    return pl.pallas_call(
        body,
        grid=(N // BR,),
        in_specs=[
            pl.BlockSpec((1, BR, H2), lambda i: (0, i, 0)),
            pl.BlockSpec((1, BR, H2), lambda i: (1, i, 0)),
            pl.BlockSpec((BR, H2), lambda i: (i, 0)),
            pl.BlockSpec((BR, 1), lambda i: (i, 0)),
            pl.BlockSpec((1, H2), lambda i: (0, 0)),
            pl.BlockSpec((H2, O), lambda i: (0, 0)),
            pl.BlockSpec((1, O), lambda i: (0, 0)),
        ],
        out_specs=pl.BlockSpec((BR, O), lambda i: (i, 0)),
        out_shape=jax.ShapeDtypeStruct((N, O), jnp.float32),
    )(agg2, agg2, hs2, dinv, b2, Wl, bl)


def kernel(x, edge_index, W1, b1, W2, b2, Wl, bl):
    N = x.shape[0]
    E = edge_index.shape[1]
    ei = edge_index.astype(jnp.int32)
    src = ei[0]
    dst = ei[1]

    deg = _make_degree(N, E)(dst)
    hs1, dinv = _mm1_call(x, W1, deg.reshape(2, N, 1))
    agg1 = _make_agg(N, W1.shape[1], E)(hs1, src, dst)
    hs2 = _mid_call(agg1, hs1, dinv, b1.reshape(1, -1), W2)
    agg2 = _make_agg(N, W2.shape[1], E)(hs2, src, dst)
    return _final_call(agg2, hs2, dinv, b2.reshape(1, -1), Wl, bl.reshape(1, -1))


# speedup vs baseline: 39.8126x; 2.2288x over previous
import functools

import jax
import jax.numpy as jnp
from jax import lax
from jax.experimental import pallas as pl
from jax.experimental.pallas import tpu as pltpu
from jax.experimental.pallas import tpu_sc as plsc

N_SC_CORES = 2
N_SUBCORES = 16
N_WORKERS = N_SC_CORES * N_SUBCORES
EDGE_BLK = 128
LANES = 16

_SC_PARAMS = pltpu.CompilerParams(use_tc_tiling_on_sc=False)


def _fill_rows(ref, rows, cols, value):
    v = jnp.full((LANES,), value, jnp.float32)
    per_row = cols // LANES

    def body(i, carry):
        r = i // per_row
        c = (i % per_row) * LANES
        ref[r, pl.ds(c, LANES)] = v
        return carry

    lax.fori_loop(0, rows * per_row, body, 0)


def _fill_flat(ref, n, value):
    v = jnp.full((LANES,), value, jnp.float32)

    def body(i, carry):
        ref[pl.ds(i * LANES, LANES)] = v
        return carry

    lax.fori_loop(0, n // LANES, body, 0)


def _make_degree(N, E):
    n_chunks = E // EDGE_BLK
    rows_pt = n_chunks // N_WORKERS
    extra = n_chunks - rows_pt * N_WORKERS
    NB = 6
    ngroups = rows_pt // NB
    assert rows_pt % NB == 0
    full_ch = 640
    tail = N - (N_SUBCORES - 1) * full_ch
    mesh = plsc.VectorSubcoreMesh(core_axis_name="c", subcore_axis_name="s")

    @functools.partial(
        pl.kernel,
        out_type=jax.ShapeDtypeStruct((N_SC_CORES * N,), jnp.float32),
        mesh=mesh,
        scratch_types=[
            pltpu.VMEM((rows_pt, EDGE_BLK), jnp.int32),
            pltpu.VMEM((EDGE_BLK,), jnp.int32),
            pltpu.VMEM((EDGE_BLK,), jnp.float32),
            pltpu.VMEM((full_ch,), jnp.float32),
            pltpu.VMEM_SHARED((N,), jnp.float32),
            pltpu.SemaphoreType.DMA,
        ],
        compiler_params=_SC_PARAMS,
    )
    def deg_kernel(dst2_hbm, out_hbm, idx_d, ex_d, ones_v, zeros_v, acc_sh, sem):
        cid = lax.axis_index("c")
        sid = lax.axis_index("s")
        wid = cid * N_SUBCORES + sid
        _fill_flat(ones_v, EDGE_BLK, 1.0)
        _fill_flat(zeros_v, full_ch, 0.0)
        pltpu.sync_copy(dst2_hbm.at[pl.ds(wid * rows_pt, rows_pt)], idx_d)

        @pl.when(wid < extra)
        def _():
            pltpu.sync_copy(dst2_hbm.at[n_chunks - extra + wid], ex_d)

        base = sid * full_ch

        @pl.when(sid < N_SUBCORES - 1)
        def _():
            pltpu.sync_copy(zeros_v, acc_sh.at[pl.ds(base, full_ch)])

        @pl.when(sid == N_SUBCORES - 1)
        def _():
            pltpu.sync_copy(zeros_v.at[pl.ds(0, tail)], acc_sh.at[pl.ds(base, tail)])

        plsc.subcore_barrier()

        def body(g, carry):
            descs = []
            for b in range(NB):
                descs.append(pltpu.async_copy(
                    ones_v, acc_sh.at[idx_d.at[g * NB + b]], sem, add=True))
            for d in descs:
                d.wait()
            return carry

        lax.fori_loop(0, ngroups, body, 0)

        @pl.when(wid < extra)
        def _():
            pltpu.sync_copy(ones_v, acc_sh.at[ex_d], add=True)

        plsc.subcore_barrier()

        @pl.when(sid < N_SUBCORES - 1)
        def _():
            pltpu.sync_copy(acc_sh.at[pl.ds(base, full_ch)], zeros_v)
            pltpu.sync_copy(zeros_v, out_hbm.at[pl.ds(cid * N + base, full_ch)])

        @pl.when(sid == N_SUBCORES - 1)
        def _():
            pltpu.sync_copy(acc_sh.at[pl.ds(base, tail)], zeros_v.at[pl.ds(0, tail)])
            pltpu.sync_copy(zeros_v.at[pl.ds(0, tail)],
                            out_hbm.at[pl.ds(cid * N + base, tail)])

    return deg_kernel


def _unpack_idx(packed_ref, j, src_ref, dst_ref):
    def body(i, carry):
        p = packed_ref[j, pl.ds(i * LANES, LANES)]
        src_ref[pl.ds(i * LANES, LANES)] = p & 0xFFFF
        dst_ref[pl.ds(i * LANES, LANES)] = lax.shift_right_logical(p, 16)
        return carry

    lax.fori_loop(0, EDGE_BLK // LANES, body, 0)


def _make_agg(N, D, E):
    n_chunks = E // EDGE_BLK
    rows_pt = n_chunks // N_WORKERS
    extra = n_chunks - rows_pt * N_WORKERS
    NB = 2 if D >= 128 else 6
    ngroups = rows_pt // NB
    assert rows_pt % NB == 0
    rpt = N // N_SUBCORES
    bch = 125
    mesh = plsc.VectorSubcoreMesh(core_axis_name="c", subcore_axis_name="s")

    @functools.partial(
        pl.kernel,
        out_type=jax.ShapeDtypeStruct((N_SC_CORES, N, D), jnp.float32),
        mesh=mesh,
        scratch_types=[
            pltpu.VMEM((rows_pt, EDGE_BLK), jnp.int32),
            pltpu.VMEM((EDGE_BLK,), jnp.int32),
            pltpu.VMEM((EDGE_BLK,), jnp.int32),
            pltpu.VMEM_SHARED((N, D), jnp.float32),
        ] + [pltpu.VMEM((EDGE_BLK, D), jnp.float32) for _ in range(NB)]
          + [pltpu.VMEM((EDGE_BLK,), jnp.int32) for _ in range(NB)]
          + [pltpu.VMEM((EDGE_BLK,), jnp.int32) for _ in range(NB)]
          + [pltpu.SemaphoreType.DMA for _ in range(NB)]
          + [pltpu.SemaphoreType.DMA],
        compiler_params=_SC_PARAMS,
    )
    def agg_kernel(hs_hbm, packed_hbm, out_hbm,
                   pidx, ex_s, ex_d, acc_sh, *rest):
        rows = rest[:NB]
        srcs = rest[NB:2 * NB]
        dsts = rest[2 * NB:3 * NB]
        sem_g = rest[3 * NB:4 * NB]
        sem_s = rest[4 * NB]
        cid = lax.axis_index("c")
        sid = lax.axis_index("s")
        wid = cid * N_SUBCORES + sid

        pltpu.sync_copy(packed_hbm.at[pl.ds(wid * rows_pt, rows_pt)], pidx)
        _fill_rows(rows[0], EDGE_BLK, D, 0.0)
        base = sid * rpt
        for k in range(rpt // bch):
            pltpu.sync_copy(rows[0].at[pl.ds(0, bch)],
                            acc_sh.at[pl.ds(base + k * bch, bch)])

        for b in range(NB):
            _unpack_idx(pidx, b, srcs[b], dsts[b])
            pltpu.async_copy(hs_hbm.at[srcs[b]], rows[b], sem_g[b])
        plsc.subcore_barrier()

        def body(g, carry):
            descs = []
            for b in range(NB):
                pltpu.make_async_copy(hs_hbm.at[srcs[b]], rows[b],
                                      sem_g[b]).wait()
                descs.append(pltpu.async_copy(
                    rows[b], acc_sh.at[dsts[b]], sem_s, add=True))
            for b in range(NB):
                descs[b].wait()

                @pl.when(g + 1 < ngroups)
                def _():
                    jn = (g + 1) * NB + b
                    _unpack_idx(pidx, jn, srcs[b], dsts[b])
                    pltpu.async_copy(hs_hbm.at[srcs[b]], rows[b], sem_g[b])
            return carry

        lax.fori_loop(0, ngroups, body, 0)

        @pl.when(wid < extra)
        def _():
            pltpu.sync_copy(packed_hbm.at[n_chunks - extra + wid], ex_s)

            def unpack_body(i, carry):
                p = ex_s[pl.ds(i * LANES, LANES)]
                ex_s[pl.ds(i * LANES, LANES)] = p & 0xFFFF
                ex_d[pl.ds(i * LANES, LANES)] = lax.shift_right_logical(p, 16)
                return carry

            lax.fori_loop(0, EDGE_BLK // LANES, unpack_body, 0)
            pltpu.async_copy(hs_hbm.at[ex_s], rows[0], sem_g[0]).wait()
            pltpu.sync_copy(rows[0], acc_sh.at[ex_d], add=True)

        plsc.subcore_barrier()
        for k in range(rpt // bch):
            pltpu.sync_copy(acc_sh.at[pl.ds(base + k * bch, bch)],
                            rows[0].at[pl.ds(0, bch)])
            pltpu.sync_copy(rows[0].at[pl.ds(0, bch)],
                            out_hbm.at[cid, pl.ds(base + k * bch, bch)])

    return agg_kernel


def _pack_call(ei2):
    C = ei2.shape[1]

    def body(e_ref, o_ref):
        o_ref[...] = jnp.left_shift(e_ref[1], 16) | e_ref[0]

    return pl.pallas_call(
        body,
        in_specs=[pl.BlockSpec((2, C, EDGE_BLK), lambda: (0, 0, 0))],
        out_specs=pl.BlockSpec((C, EDGE_BLK), lambda: (0, 0)),
        out_shape=jax.ShapeDtypeStruct((C, EDGE_BLK), jnp.int32),
    )(ei2)


def _mm1_call(x, W1, deg):
    N, C = x.shape
    H = W1.shape[1]
    BR = 1000

    def body(x_ref, w_ref, deg_ref, hs_ref, dinv_ref):
        d = lax.rsqrt(deg_ref[0] + deg_ref[1] + 1.0)
        h = jnp.dot(x_ref[...], w_ref[...], preferred_element_type=jnp.float32)
        hs_ref[...] = h * d
        dinv_ref[...] = d

    return pl.pallas_call(
        body,
        grid=(N // BR,),
        in_specs=[
            pl.BlockSpec((BR, C), lambda i: (i, 0)),
            pl.BlockSpec((C, H), lambda i: (0, 0)),
            pl.BlockSpec((2, BR, 1), lambda i: (0, i, 0)),
        ],
        out_specs=[
            pl.BlockSpec((BR, H), lambda i: (i, 0)),
            pl.BlockSpec((BR, 1), lambda i: (i, 0)),
        ],
        out_shape=[
            jax.ShapeDtypeStruct((N, H), jnp.float32),
            jax.ShapeDtypeStruct((N, 1), jnp.float32),
        ],
    )(x, W1, deg)


def _mid_call(agg1, hs1, dinv, b1, W2):
    N, H = hs1.shape
    H2 = W2.shape[1]
    BR = 1000

    def body(a0_ref, a1_ref, hs_ref, d_ref, b_ref, w_ref, o_ref):
        a = a0_ref[0] + a1_ref[0]
        d = d_ref[...]
        z = d * (a + hs_ref[...]) + b_ref[...]
        r = jnp.maximum(z, 0.0)
        o_ref[...] = d * jnp.dot(r, w_ref[...], preferred_element_type=jnp.float32)

    return pl.pallas_call(
        body,
        grid=(N // BR,),
        in_specs=[
            pl.BlockSpec((1, BR, H), lambda i: (0, i, 0)),
            pl.BlockSpec((1, BR, H), lambda i: (1, i, 0)),
            pl.BlockSpec((BR, H), lambda i: (i, 0)),
            pl.BlockSpec((BR, 1), lambda i: (i, 0)),
            pl.BlockSpec((1, H), lambda i: (0, 0)),
            pl.BlockSpec((H, H2), lambda i: (0, 0)),
        ],
        out_specs=pl.BlockSpec((BR, H2), lambda i: (i, 0)),
        out_shape=jax.ShapeDtypeStruct((N, H2), jnp.float32),
    )(agg1, agg1, hs1, dinv, b1, W2)


def _final_call(agg2, hs2, dinv, b2, Wl, bl):
    N, H2 = hs2.shape
    O = Wl.shape[1]
    BR = 1000

    def body(a0_ref, a1_ref, hs_ref, d_ref, b_ref, w_ref, bl_ref, o_ref):
        a = a0_ref[0] + a1_ref[0]
        d = d_ref[...]
        z = d * (a + hs_ref[...]) + b_ref[...]
        logits = jnp.dot(z, w_ref[...], preferred_element_type=jnp.float32)
        logits = logits + bl_ref[...]
        m = jnp.max(logits, axis=1, keepdims=True)
        lse = jnp.log(jnp.sum(jnp.exp(logits - m), axis=1, keepdims=True)) + m
        o_ref[...] = logits - lse

    return pl.pallas_call(
        body,
        grid=(N // BR,),
        in_specs=[
            pl.BlockSpec((1, BR, H2), lambda i: (0, i, 0)),
            pl.BlockSpec((1, BR, H2), lambda i: (1, i, 0)),
            pl.BlockSpec((BR, H2), lambda i: (i, 0)),
            pl.BlockSpec((BR, 1), lambda i: (i, 0)),
            pl.BlockSpec((1, H2), lambda i: (0, 0)),
            pl.BlockSpec((H2, O), lambda i: (0, 0)),
            pl.BlockSpec((1, O), lambda i: (0, 0)),
        ],
        out_specs=pl.BlockSpec((BR, O), lambda i: (i, 0)),
        out_shape=jax.ShapeDtypeStruct((N, O), jnp.float32),
    )(agg2, agg2, hs2, dinv, b2, Wl, bl)


def kernel(x, edge_index, W1, b1, W2, b2, Wl, bl):
    N = x.shape[0]
    E = edge_index.shape[1]
    ei = edge_index.astype(jnp.int32)
    ei2 = ei.reshape(2, E // EDGE_BLK, EDGE_BLK)
    dst2 = ei2[1]
    packed = _pack_call(ei2)

    deg = _make_degree(N, E)(dst2)
    hs1, dinv = _mm1_call(x, W1, deg.reshape(2, N, 1))
    agg1 = _make_agg(N, W1.shape[1], E)(hs1, packed)
    hs2 = _mid_call(agg1, hs1, dinv, b1.reshape(1, -1), W2)
    agg2 = _make_agg(N, W2.shape[1], E)(hs2, packed)
    return _final_call(agg2, hs2, dinv, b2.reshape(1, -1), Wl, bl.reshape(1, -1))
